# baseline (speedup 1.0000x reference)
import functools

import jax
import jax.numpy as jnp
from jax import lax
from jax.experimental import pallas as pl
from jax.experimental.pallas import tpu as pltpu
from jax.experimental.pallas import tpu_sc as plsc

_SMOOTH = 1e-5
_C = 19
_NC, _NS, _NL = 2, 16, 16
_NW = _NC * _NS
_CH = 2048
_LN2 = 0.6931471805599453


def _log16(x):
    xi = plsc.bitcast(x, jnp.int32)
    e = (xi >> 23) - 127
    m = plsc.bitcast((xi & 0x007FFFFF) | 0x3F800000, jnp.float32)
    big = m > 1.4142135
    m = jnp.where(big, m * 0.5, m)
    e = jnp.where(big, e + 1, e)
    t = (m - 1.0) / (m + 1.0)
    t2 = t * t
    p = 2.0 + t2 * (2.0 / 3.0 + t2 * (2.0 / 5.0 + t2 * (2.0 / 7.0)))
    return e.astype(jnp.float32) * _LN2 + t * p


def _sc_body(lg_hbm, tg_hbm, out_hbm, buf, tbuf, accv):
    n_chunks = tg_hbm.shape[0] // _CH
    cpw = n_chunks // _NW
    cpb = lg_hbm.shape[2] // _CH
    a_coef = 1.0 - _SMOOTH - _SMOOTH / (_C - 1)
    b_coef = _SMOOTH / (_C - 1)
    wid = lax.axis_index("s") * _NC + lax.axis_index("c")

    def chunk_body(i, acc):
        cid = wid * cpw + i
        b = cid // cpb
        off = (cid % cpb) * _CH
        pltpu.sync_copy(lg_hbm.at[b, :, pl.ds(off, _CH)], buf)
        pltpu.sync_copy(tg_hbm.at[pl.ds(cid * _CH, _CH)], tbuf)

        def px_body(k, acc):
            w0 = k * _NL
            t16 = tbuf[pl.ds(w0, _NL)]
            w16 = lax.broadcasted_iota(jnp.int32, (_NL,), 0) + w0
            lgt = plsc.load_gather(buf, [t16, w16])
            s = buf[0, pl.ds(w0, _NL)]
            for c in range(1, _C):
                s = s + buf[c, pl.ds(w0, _NL)]
            pt = a_coef * lgt + b_coef * s + _SMOOTH
            om = 1.0 - pt
            return acc + om * om * _log16(pt)

        return lax.fori_loop(0, _CH // _NL, px_body, acc)

    acc = lax.fori_loop(0, cpw, chunk_body, jnp.zeros((_NL,), jnp.float32))
    accv[...] = acc
    pltpu.sync_copy(accv, out_hbm.at[wid])


def kernel(logit, target):
    B, C, H, W = logit.shape
    lg = logit.reshape(B, C, H * W)
    tg = target.astype(jnp.int32).reshape(B * H * W)
    mesh = plsc.VectorSubcoreMesh(core_axis_name="c", subcore_axis_name="s")
    partials = pl.kernel(
        _sc_body,
        out_type=jax.ShapeDtypeStruct((_NW, _NL), jnp.float32),
        mesh=mesh,
        scratch_types=[
            pltpu.VMEM((_C, _CH), jnp.float32),
            pltpu.VMEM((_CH,), jnp.int32),
            pltpu.VMEM((_NL,), jnp.float32),
        ],
        compiler_params=pltpu.CompilerParams(needs_layout_passes=False),
    )(lg, tg)
    return -jnp.sum(partials) / (B * H * W)


# speedup vs baseline: 1.2566x; 1.2566x over previous
import jax
import jax.numpy as jnp
from jax import lax
from jax.experimental import pallas as pl
from jax.experimental.pallas import tpu as pltpu
from jax.experimental.pallas import tpu_sc as plsc

_SMOOTH = 1e-5
_C = 19
_NC, _NS, _NL = 2, 16, 16
_NW = _NC * _NS
_CH = 2048
_LN2 = 0.6931471805599453


def _log16(x):
    xi = plsc.bitcast(x, jnp.int32)
    e = (xi >> 23) - 127
    m = plsc.bitcast((xi & 0x007FFFFF) | 0x3F800000, jnp.float32)
    big = m > 1.4142135
    m = jnp.where(big, m * 0.5, m)
    e = jnp.where(big, e + 1, e)
    t = (m - 1.0) / (m + 1.0)
    t2 = t * t
    p = 2.0 + t2 * (2.0 / 3.0 + t2 * (2.0 / 5.0 + t2 * (2.0 / 7.0)))
    return e.astype(jnp.float32) * _LN2 + t * p


def _sc_body(lg_hbm, tg_hbm, out_hbm, buf0, buf1, tb0, tb1, accv, sem0, sem1):
    n_chunks = tg_hbm.shape[0] // _CH
    cpw = n_chunks // _NW
    cpb = lg_hbm.shape[2] // _CH
    a_coef = 1.0 - _SMOOTH - _SMOOTH / (_C - 1)
    b_coef = _SMOOTH / (_C - 1)
    wid = lax.axis_index("s") * _NC + lax.axis_index("c")
    bufs, tbs, sems = (buf0, buf1), (tb0, tb1), (sem0, sem1)
    lane = lax.broadcasted_iota(jnp.int32, (_NL,), 0)

    def issue(i, q):
        cid = wid * cpw + i
        b = cid // cpb
        off = (cid % cpb) * _CH
        pltpu.async_copy(lg_hbm.at[b, :, pl.ds(off, _CH)], bufs[q], sems[q])
        pltpu.async_copy(tg_hbm.at[pl.ds(cid * _CH, _CH)], tbs[q], sems[q])

    def drain(q):
        pltpu.make_async_copy(
            lg_hbm.at[0, :, pl.ds(0, _CH)], bufs[q], sems[q]).wait()
        pltpu.make_async_copy(
            tg_hbm.at[pl.ds(0, _CH)], tbs[q], sems[q]).wait()

    def px16(buf, tbuf, w0, acc):
        t16 = tbuf[pl.ds(w0, _NL)]
        lgt = plsc.load_gather(buf, [t16, lane + w0])
        rows = [buf[c, pl.ds(w0, _NL)] for c in range(_C)]
        while len(rows) > 1:
            nxt = [rows[2 * j] + rows[2 * j + 1] for j in range(len(rows) // 2)]
            if len(rows) % 2:
                nxt.append(rows[-1])
            rows = nxt
        pt = a_coef * lgt + (b_coef * rows[0] + _SMOOTH)
        om = 1.0 - pt
        return acc + om * om * _log16(pt)

    def pair_body(j, acc):
        for p in (0, 1):
            i = j * 2 + p

            @pl.when(i + 1 < cpw)
            def _():
                issue(i + 1, 1 - p)

            drain(p)

            def k_body(k, acc):
                w0 = k * (2 * _NL)
                acc = px16(bufs[p], tbs[p], w0, acc)
                return px16(bufs[p], tbs[p], w0 + _NL, acc)

            acc = lax.fori_loop(0, _CH // (2 * _NL), k_body, acc)
        return acc

    issue(0, 0)
    acc = lax.fori_loop(0, cpw // 2, pair_body,
                        jnp.zeros((_NL,), jnp.float32))
    accv[...] = acc
    pltpu.sync_copy(accv, out_hbm.at[wid])


def kernel(logit, target):
    B, C, H, W = logit.shape
    lg = logit.reshape(B, C, H * W)
    tg = target.astype(jnp.int32).reshape(B * H * W)
    mesh = plsc.VectorSubcoreMesh(core_axis_name="c", subcore_axis_name="s")
    partials = pl.kernel(
        _sc_body,
        out_type=jax.ShapeDtypeStruct((_NW, _NL), jnp.float32),
        mesh=mesh,
        scratch_types=[
            pltpu.VMEM((_C, _CH), jnp.float32),
            pltpu.VMEM((_C, _CH), jnp.float32),
            pltpu.VMEM((_CH,), jnp.int32),
            pltpu.VMEM((_CH,), jnp.int32),
            pltpu.VMEM((_NL,), jnp.float32),
            pltpu.SemaphoreType.DMA,
            pltpu.SemaphoreType.DMA,
        ],
        compiler_params=pltpu.CompilerParams(needs_layout_passes=False),
    )(lg, tg)
    return -jnp.sum(partials) / (B * H * W)


# speedup vs baseline: 1.2596x; 1.0024x over previous
import jax
import jax.numpy as jnp
from jax import lax
from jax.experimental import pallas as pl
from jax.experimental.pallas import tpu as pltpu
from jax.experimental.pallas import tpu_sc as plsc

_SMOOTH = 1e-5
_C = 19
_NC, _NS, _NL = 2, 16, 16
_NW = _NC * _NS
_CH = 2048
_LN2 = 0.6931471805599453


def _log16(x):
    xi = plsc.bitcast(x, jnp.int32)
    e = (xi >> 23) - 127
    m = plsc.bitcast((xi & 0x007FFFFF) | 0x3F800000, jnp.float32)
    big = m > 1.4142135
    m = jnp.where(big, m * 0.5, m)
    e = jnp.where(big, e + 1, e)
    t = (m - 1.0) / (m + 1.0)
    t2 = t * t
    p = 2.0 + t2 * (2.0 / 3.0 + t2 * (2.0 / 5.0 + t2 * (2.0 / 7.0)))
    return e.astype(jnp.float32) * _LN2 + t * p


def _sc_body(lg_hbm, tg_hbm, out_hbm, buf0, buf1, tb0, tb1, accv, sem0, sem1):
    n_chunks = tg_hbm.shape[0] // _CH
    cpw = n_chunks // _NW
    cpb = lg_hbm.shape[2] // _CH
    a_coef = 1.0 - _SMOOTH - _SMOOTH / (_C - 1)
    b_coef = _SMOOTH / (_C - 1)
    wid = lax.axis_index("s") * _NC + lax.axis_index("c")
    bufs, tbs, sems = (buf0, buf1), (tb0, tb1), (sem0, sem1)
    lane = lax.broadcasted_iota(jnp.int32, (_NL,), 0)

    row_split = (0, 8, 16, _C)

    def issue(i, q):
        cid = wid * cpw + i
        b = cid // cpb
        off = (cid % cpb) * _CH
        for r0, r1 in zip(row_split[:-1], row_split[1:]):
            pltpu.async_copy(
                lg_hbm.at[b, pl.ds(r0, r1 - r0), pl.ds(off, _CH)],
                bufs[q].at[pl.ds(r0, r1 - r0)], sems[q])
        pltpu.async_copy(tg_hbm.at[pl.ds(cid * _CH, _CH)], tbs[q], sems[q])

    def drain(q):
        for r0, r1 in zip(row_split[:-1], row_split[1:]):
            pltpu.make_async_copy(
                lg_hbm.at[0, pl.ds(r0, r1 - r0), pl.ds(0, _CH)],
                bufs[q].at[pl.ds(r0, r1 - r0)], sems[q]).wait()
        pltpu.make_async_copy(
            tg_hbm.at[pl.ds(0, _CH)], tbs[q], sems[q]).wait()

    def px16(buf, tbuf, w0, acc):
        t16 = tbuf[pl.ds(w0, _NL)]
        lgt = plsc.load_gather(buf, [t16, lane + w0])
        rows = [buf[c, pl.ds(w0, _NL)] for c in range(_C)]
        while len(rows) > 1:
            nxt = [rows[2 * j] + rows[2 * j + 1] for j in range(len(rows) // 2)]
            if len(rows) % 2:
                nxt.append(rows[-1])
            rows = nxt
        pt = a_coef * lgt + (b_coef * rows[0] + _SMOOTH)
        om = 1.0 - pt
        return acc + om * om * _log16(pt)

    def pair_body(j, acc):
        for p in (0, 1):
            i = j * 2 + p

            @pl.when(i + 1 < cpw)
            def _():
                issue(i + 1, 1 - p)

            drain(p)

            def k_body(k, acc):
                w0 = k * (2 * _NL)
                acc = px16(bufs[p], tbs[p], w0, acc)
                return px16(bufs[p], tbs[p], w0 + _NL, acc)

            acc = lax.fori_loop(0, _CH // (2 * _NL), k_body, acc)
        return acc

    issue(0, 0)
    acc = lax.fori_loop(0, cpw // 2, pair_body,
                        jnp.zeros((_NL,), jnp.float32))
    accv[...] = acc
    pltpu.sync_copy(accv, out_hbm.at[wid])


def kernel(logit, target):
    B, C, H, W = logit.shape
    lg = logit.reshape(B, C, H * W)
    tg = target.astype(jnp.int32).reshape(B * H * W)
    mesh = plsc.VectorSubcoreMesh(core_axis_name="c", subcore_axis_name="s")
    partials = pl.kernel(
        _sc_body,
        out_type=jax.ShapeDtypeStruct((_NW, _NL), jnp.float32),
        mesh=mesh,
        scratch_types=[
            pltpu.VMEM((_C, _CH), jnp.float32),
            pltpu.VMEM((_C, _CH), jnp.float32),
            pltpu.VMEM((_CH,), jnp.int32),
            pltpu.VMEM((_CH,), jnp.int32),
            pltpu.VMEM((_NL,), jnp.float32),
            pltpu.SemaphoreType.DMA,
            pltpu.SemaphoreType.DMA,
        ],
        compiler_params=pltpu.CompilerParams(needs_layout_passes=False),
    )(lg, tg)
    return -jnp.sum(partials) / (B * H * W)
